# Initial kernel scaffold; baseline (speedup 1.0000x reference)
import jax
import jax.numpy as jnp
from jax.experimental import pallas as pl


def kernel(u, un_s, t):
    raise NotImplementedError("write your pallas kernel here")


# speedup vs baseline: 11.3117x; 11.3117x over previous
import functools
import math

import jax
import jax.numpy as jnp
from jax.experimental import pallas as pl

_LOG_SLOPE = math.log(0.001)
_N = 256
_ROWS = 16384
_COLS = 512
_BLK_ROWS = 1024


def _affine_kernel(u_ref, s_ref, t_ref, x_ref, d_ref):
    logs = s_ref[0, :] / (1.0 + jnp.abs(s_ref[0, :] / _LOG_SLOPE))
    scale = jnp.exp(logs)
    x_ref[:, :_N] = u_ref[:, :_N] * scale[None, :] + t_ref[0, :][None, :]
    x_ref[:, _N:] = u_ref[:, _N:]
    d_ref[:] = jnp.full((_BLK_ROWS,), jnp.sum(logs), dtype=jnp.float32)


@jax.jit
def kernel(u, un_s, t):
    grid = _ROWS // _BLK_ROWS
    x, logd = pl.pallas_call(
        _affine_kernel,
        grid=(grid,),
        in_specs=[
            pl.BlockSpec((_BLK_ROWS, _COLS), lambda i: (i, 0)),
            pl.BlockSpec((1, _N), lambda i: (0, 0)),
            pl.BlockSpec((1, _N), lambda i: (0, 0)),
        ],
        out_specs=[
            pl.BlockSpec((_BLK_ROWS, _COLS), lambda i: (i, 0)),
            pl.BlockSpec((_BLK_ROWS,), lambda i: (i,)),
        ],
        out_shape=[
            jax.ShapeDtypeStruct((_ROWS, _COLS), jnp.float32),
            jax.ShapeDtypeStruct((_ROWS,), jnp.float32),
        ],
    )(u, un_s.reshape(1, _N), t.reshape(1, _N))
    return (x, logd)


# speedup vs baseline: 12.5028x; 1.1053x over previous
import functools
import math

import jax
import jax.numpy as jnp
from jax.experimental import pallas as pl

_LOG_SLOPE = math.log(0.001)
_N = 256
_ROWS = 16384
_COLS = 512
_BLK_ROWS = 2048


def _affine_kernel(u_ref, s_ref, t_ref, x_ref, d_ref):
    logs = s_ref[0, :] / (1.0 + jnp.abs(s_ref[0, :] / _LOG_SLOPE))
    scale = jnp.exp(logs)
    x_ref[:, :_N] = u_ref[:, :_N] * scale[None, :] + t_ref[0, :][None, :]
    x_ref[:, _N:] = u_ref[:, _N:]
    d_ref[:] = jnp.full((_BLK_ROWS,), jnp.sum(logs), dtype=jnp.float32)


@jax.jit
def kernel(u, un_s, t):
    grid = _ROWS // _BLK_ROWS
    x, logd = pl.pallas_call(
        _affine_kernel,
        grid=(grid,),
        in_specs=[
            pl.BlockSpec((_BLK_ROWS, _COLS), lambda i: (i, 0)),
            pl.BlockSpec((1, _N), lambda i: (0, 0)),
            pl.BlockSpec((1, _N), lambda i: (0, 0)),
        ],
        out_specs=[
            pl.BlockSpec((_BLK_ROWS, _COLS), lambda i: (i, 0)),
            pl.BlockSpec((_BLK_ROWS,), lambda i: (i,)),
        ],
        out_shape=[
            jax.ShapeDtypeStruct((_ROWS, _COLS), jnp.float32),
            jax.ShapeDtypeStruct((_ROWS,), jnp.float32),
        ],
    )(u, un_s.reshape(1, _N), t.reshape(1, _N))
    return (x, logd)


# speedup vs baseline: 13.4132x; 1.0728x over previous
import functools
import math

import jax
import jax.numpy as jnp
from jax.experimental import pallas as pl

_LOG_SLOPE = math.log(0.001)
_N = 256
_ROWS = 16384
_COLS = 512
_BLK_ROWS = 4096


def _affine_kernel(u_ref, s_ref, t_ref, x_ref, d_ref):
    logs = s_ref[0, :] / (1.0 + jnp.abs(s_ref[0, :] / _LOG_SLOPE))
    scale = jnp.exp(logs)
    x_ref[:, :_N] = u_ref[:, :_N] * scale[None, :] + t_ref[0, :][None, :]
    x_ref[:, _N:] = u_ref[:, _N:]
    d_ref[:] = jnp.full((_BLK_ROWS,), jnp.sum(logs), dtype=jnp.float32)


@jax.jit
def kernel(u, un_s, t):
    grid = _ROWS // _BLK_ROWS
    x, logd = pl.pallas_call(
        _affine_kernel,
        grid=(grid,),
        in_specs=[
            pl.BlockSpec((_BLK_ROWS, _COLS), lambda i: (i, 0)),
            pl.BlockSpec((1, _N), lambda i: (0, 0)),
            pl.BlockSpec((1, _N), lambda i: (0, 0)),
        ],
        out_specs=[
            pl.BlockSpec((_BLK_ROWS, _COLS), lambda i: (i, 0)),
            pl.BlockSpec((_BLK_ROWS,), lambda i: (i,)),
        ],
        out_shape=[
            jax.ShapeDtypeStruct((_ROWS, _COLS), jnp.float32),
            jax.ShapeDtypeStruct((_ROWS,), jnp.float32),
        ],
    )(u, un_s.reshape(1, _N), t.reshape(1, _N))
    return (x, logd)
